# baseline (speedup 1.0000x reference)
import functools

import jax
import jax.numpy as jnp
from jax import lax
from jax.experimental import pallas as pl
from jax.experimental.pallas import tpu as pltpu
from jax.experimental.pallas import tpu_sc as plsc

V = 512
E = 128
P = 8
BT = 128
S = 256
NC, NS, L = 2, 16, 16
NW = NC * NS
BT_PER_W = BT // NW

_mesh = plsc.VectorSubcoreMesh(core_axis_name="c", subcore_axis_name="s")


@functools.partial(
    pl.kernel,
    mesh=_mesh,
    compiler_params=pltpu.CompilerParams(needs_layout_passes=False),
    out_type=jax.ShapeDtypeStruct((BT, S * E), jnp.float32),
    scratch_types=[
        pltpu.VMEM((V * E,), jnp.float32),
        pltpu.VMEM((P * S,), jnp.int32),
        pltpu.VMEM((S * E,), jnp.float32),
    ],
)
def _embed_sc(x_hbm, w_hbm, out_hbm, w_v, x_v, o_v):
    wid = lax.axis_index("s") * NC + lax.axis_index("c")
    pltpu.sync_copy(w_hbm, w_v)

    def bt_body(i, carry):
        bt = wid * BT_PER_W + i
        pltpu.sync_copy(x_hbm.at[bt], x_v)

        @plsc.parallel_loop(0, S // L, step=1, unroll=1)
        def g_body(g):
            s0 = g * L
            bases = [x_v[pl.ds(p * S + s0, L)] * E for p in range(P)]
            for j in range(L):
                rows = [bases[p][j] for p in range(P)]
                sb = (s0 + j) * E
                for dc in range(E // L):
                    o = dc * L
                    v0 = w_v[pl.ds(rows[0] + o, L)] + w_v[pl.ds(rows[1] + o, L)]
                    v1 = w_v[pl.ds(rows[2] + o, L)] + w_v[pl.ds(rows[3] + o, L)]
                    v2 = w_v[pl.ds(rows[4] + o, L)] + w_v[pl.ds(rows[5] + o, L)]
                    v3 = w_v[pl.ds(rows[6] + o, L)] + w_v[pl.ds(rows[7] + o, L)]
                    o_v[pl.ds(sb + o, L)] = (v0 + v1) + (v2 + v3)

        pltpu.sync_copy(o_v, out_hbm.at[bt])
        return carry

    lax.fori_loop(0, BT_PER_W, bt_body, 0)


def kernel(x, W):
    xt = x.astype(jnp.int32).reshape(BT, P * S)
    wf = W.reshape(V * E)
    out = _embed_sc(xt, wf)
    out = out.reshape(16, 8, 16, 16, E)
    return jnp.transpose(out, (0, 1, 4, 2, 3))
